# baseline (speedup 1.0000x reference)
import jax
import jax.numpy as jnp
from jax.experimental import pallas as pl
from jax.experimental.pallas import tpu as pltpu

_G = 64
_T = 12
_K = _G * _G
_BK = 512
_NBLK = _K // _BK


def _reduce_kernel(a_ref, od_ref, harm_ref, mu0_ref, out_ref, acc_ref):
    k = pl.program_id(0)

    @pl.when(k == 0)
    def _init():
        acc_ref[...] = jnp.zeros_like(acc_ref)

    acc_ref[...] += jnp.dot(a_ref[...], od_ref[...],
                            preferred_element_type=jnp.float32)

    @pl.when(k == _NBLK - 1)
    def _finish():
        harm = harm_ref[:, 0:1]
        mu0_flat = mu0_ref[...]
        base = acc_ref[0:1, :]
        colsum = acc_ref[1:2, :]
        out_ref[:, 0, :] = mu0_flat + harm
        out_ref[:, 1, :] = base + harm * colsum


def kernel(t_array, mu0, a_k, b_k, od_matrix):
    mu0_flat = mu0.reshape(1, _K).astype(jnp.float32)
    od2 = od_matrix.reshape(_K, _K)

    t_norm = 2.0 * jnp.pi * t_array / 120.0
    harm = (a_k[0] * jnp.sin(t_norm) + b_k[0] * jnp.cos(t_norm)
            + a_k[1] * jnp.sin(2.0 * t_norm) + b_k[1] * jnp.cos(2.0 * t_norm))
    harm2 = jnp.broadcast_to(harm[:, None], (_T, 128)).astype(jnp.float32)

    a_mat = jnp.concatenate(
        [mu0_flat,
         jnp.ones((1, _K), jnp.float32),
         jnp.zeros((6, _K), jnp.float32)], axis=0)

    out = pl.pallas_call(
        _reduce_kernel,
        grid=(_NBLK,),
        in_specs=[
            pl.BlockSpec((8, _BK), lambda k: (0, k)),
            pl.BlockSpec((_BK, _K), lambda k: (k, 0)),
            pl.BlockSpec((_T, 128), lambda k: (0, 0)),
            pl.BlockSpec((1, _K), lambda k: (0, 0)),
        ],
        out_specs=pl.BlockSpec((_T, 2, _K), lambda k: (0, 0, 0)),
        out_shape=jax.ShapeDtypeStruct((_T, 2, _K), jnp.float32),
        scratch_shapes=[pltpu.VMEM((8, _K), jnp.float32)],
        compiler_params=pltpu.CompilerParams(
            dimension_semantics=("arbitrary",)),
    )(a_mat, od2, harm2, mu0_flat)

    return out.reshape(_T, 2, _G, _G)
